# baseline (speedup 1.0000x reference)
import functools

import jax
import jax.numpy as jnp
from jax import lax
from jax.experimental import pallas as pl
from jax.experimental.pallas import tpu as pltpu
from jax.experimental.pallas import tpu_sc as plsc

N = 10000
E = 320000
D = 128

NC = 2
NS = 16
CHUNK = 128
G = 16
NG = 5
TOT_CH = NC * NS * G * NG
PE = TOT_CH * CHUNK
AGG_ROWS = 10240
ZROWS = AGG_ROWS // NS
TAIL = 24

_sc_mesh = plsc.VectorSubcoreMesh(core_axis_name="c", subcore_axis_name="s")


@functools.partial(
    pl.kernel,
    out_type=jax.ShapeDtypeStruct((NC, AGG_ROWS, D), jnp.float32),
    mesh=_sc_mesh,
    scratch_types=[
        pltpu.VMEM((G, CHUNK), jnp.int32),
        pltpu.VMEM((G, CHUNK), jnp.int32),
        pltpu.VMEM((CHUNK, D), jnp.float32),
        pltpu.VMEM((CHUNK, D), jnp.float32),
        pltpu.VMEM_SHARED((AGG_ROWS, D), jnp.float32),
        pltpu.SemaphoreType.DMA,
        pltpu.SemaphoreType.DMA,
    ],
)
def _sc_agg(h_hbm, src_hbm, dst_hbm, out_hbm, src_v, dst_v, rows_a, rows_b,
            agg, sem_a, sem_b):
    c = lax.axis_index("c")
    s = lax.axis_index("s")
    tile_chunk_base = (c * NS + s) * (G * NG)

    zv = jnp.zeros((16,), jnp.float32)

    with jax.named_scope("agg_zero"):
        @pl.loop(0, CHUNK)
        def _zero(i):
            for j in range(D // 16):
                rows_a[i, pl.ds(j * 16, 16)] = zv

        for k in range(ZROWS // CHUNK):
            pltpu.sync_copy(rows_a, agg.at[pl.ds(s * ZROWS + k * CHUNK, CHUNK)])
        plsc.subcore_barrier()

    def _start_gather(j, buf, sem):
        pltpu.async_copy(h_hbm.at[src_v.at[j]], buf, sem)

    def _wait_gather(buf, sem):
        pltpu.make_async_copy(h_hbm.at[pl.ds(0, CHUNK)], buf, sem).wait()

    with jax.named_scope("agg_edges"):
        for g in range(NG):
            gbase = tile_chunk_base + g * G
            pltpu.sync_copy(src_hbm.at[pl.ds(gbase, G)], src_v)
            pltpu.sync_copy(dst_hbm.at[pl.ds(gbase, G)], dst_v)
            _start_gather(0, rows_a, sem_a)

            @pl.loop(0, G, step=2)
            def _edges(j):
                _start_gather(j + 1, rows_b, sem_b)
                _wait_gather(rows_a, sem_a)
                pltpu.sync_copy(rows_a, agg.at[dst_v.at[j]], add=True)

                @pl.when(j + 2 < G)
                def _():
                    _start_gather(j + 2, rows_a, sem_a)

                _wait_gather(rows_b, sem_b)
                pltpu.sync_copy(rows_b, agg.at[dst_v.at[j + 1]], add=True)

    with jax.named_scope("agg_wout"):
        plsc.subcore_barrier()
        pltpu.sync_copy(
            agg.at[pl.ds(s * ZROWS, ZROWS)],
            out_hbm.at[c, pl.ds(s * ZROWS, ZROWS)],
        )

    with jax.named_scope("agg_tail"):
        @pl.loop(0, TAIL)
        def _spin(i):
            pltpu.sync_copy(h_hbm.at[pl.ds(0, 8)], rows_a.at[pl.ds(0, 8)])


BLK = 1000


def _mlp_body(h_ref, p0_ref, p1_ref, wa_ref, ba_ref, wb_ref, bb_ref, o_ref):
    t = h_ref[...] + p0_ref[...] + p1_ref[...]
    t = jnp.dot(t, wa_ref[...], preferred_element_type=jnp.float32) + ba_ref[...]
    o_ref[...] = jnp.dot(t, wb_ref[...], preferred_element_type=jnp.float32) + bb_ref[...]


_row_spec = pl.BlockSpec((BLK, D), lambda i: (i, 0))
_w_spec = pl.BlockSpec((D, D), lambda i: (0, 0))
_b_spec = pl.BlockSpec((1, D), lambda i: (0, 0))

_mlp = pl.pallas_call(
    _mlp_body,
    grid=(N // BLK,),
    in_specs=[_row_spec, _row_spec, _row_spec, _w_spec, _b_spec, _w_spec, _b_spec],
    out_specs=_row_spec,
    out_shape=jax.ShapeDtypeStruct((N, D), jnp.float32),
)


def _mlp_head_body(h_ref, p0_ref, p1_ref, wa_ref, ba_ref, wb_ref, bb_ref,
                   wl1_ref, bl1_ref, wl3_ref, bl3_ref, o_ref):
    t = h_ref[...] + p0_ref[...] + p1_ref[...]
    t = jnp.dot(t, wa_ref[...], preferred_element_type=jnp.float32) + ba_ref[...]
    t = jnp.dot(t, wb_ref[...], preferred_element_type=jnp.float32) + bb_ref[...]
    g = jnp.maximum(
        jnp.dot(t, wl1_ref[...], preferred_element_type=jnp.float32) + bl1_ref[...],
        0.0,
    )
    o_ref[...] = jnp.dot(g, wl3_ref[...], preferred_element_type=jnp.float32) + bl3_ref[...]


_mlp_head = pl.pallas_call(
    _mlp_head_body,
    grid=(N // BLK,),
    in_specs=[
        _row_spec, _row_spec, _row_spec,
        _w_spec, _b_spec, _w_spec, _b_spec,
        _w_spec, _b_spec,
        pl.BlockSpec((D, 1), lambda i: (0, 0)),
        pl.BlockSpec((1, 1), lambda i: (0, 0)),
    ],
    out_specs=pl.BlockSpec((BLK, 1), lambda i: (i, 0)),
    out_shape=jax.ShapeDtypeStruct((N, 1), jnp.float32),
)


def kernel(x, edge_index, W1a, b1a, W1b, b1b, W2a, b2a, W2b, b2b, Wl1, bl1, Wl3, bl3):
    src = edge_index[0].astype(jnp.int32)
    dst = edge_index[1].astype(jnp.int32)
    pad = PE - E
    src_p = jnp.concatenate([src, jnp.zeros((pad,), jnp.int32)]).reshape(-1, CHUNK)
    dst_p = jnp.concatenate([dst, jnp.full((pad,), N, jnp.int32)]).reshape(-1, CHUNK)

    b1a_r = b1a.reshape(1, D)
    b1b_r = b1b.reshape(1, D)
    b2a_r = b2a.reshape(1, D)
    b2b_r = b2b.reshape(1, D)
    bl1_r = bl1.reshape(1, D)
    bl3_r = bl3.reshape(1, 1)

    p = _sc_agg(x, src_p, dst_p)
    h1 = _mlp(x, p[0], p[1], W1a, b1a_r, W1b, b1b_r)
    p = _sc_agg(h1, src_p, dst_p)
    out = _mlp_head(h1, p[0], p[1], W2a, b2a_r, W2b, b2b_r, Wl1, bl1_r, Wl3, bl3_r)
    return out


# speedup vs baseline: 3.5393x; 3.5393x over previous
import functools

import jax
import jax.numpy as jnp
from jax import lax
from jax.experimental import pallas as pl
from jax.experimental.pallas import tpu as pltpu
from jax.experimental.pallas import tpu_sc as plsc

N = 10000
E = 320000
D = 128

NC = 2
NS = 16
CHUNK = 128
G = 16
NG = 5
TOT_CH = NC * NS * G * NG
PE = TOT_CH * CHUNK
AGG_ROWS = 10240
ZROWS = AGG_ROWS // NS

_sc_mesh = plsc.VectorSubcoreMesh(core_axis_name="c", subcore_axis_name="s")


@functools.partial(
    pl.kernel,
    out_type=jax.ShapeDtypeStruct((NC, AGG_ROWS, D), jnp.float32),
    mesh=_sc_mesh,
    scratch_types=[
        pltpu.VMEM((G, CHUNK), jnp.int32),
        pltpu.VMEM((G, CHUNK), jnp.int32),
        pltpu.VMEM((CHUNK, D), jnp.float32),
        pltpu.VMEM((CHUNK, D), jnp.float32),
        pltpu.VMEM_SHARED((AGG_ROWS, D), jnp.float32),
        pltpu.SemaphoreType.DMA,
        pltpu.SemaphoreType.DMA,
    ],
)
def _sc_agg(h_hbm, src_hbm, dst_hbm, out_hbm, src_v, dst_v, rows_a, rows_b,
            agg, sem_a, sem_b):
    c = lax.axis_index("c")
    s = lax.axis_index("s")
    tile_chunk_base = (c * NS + s) * (G * NG)

    zv = jnp.zeros((16,), jnp.float32)

    with jax.named_scope("agg_zero"):
        @pl.loop(0, CHUNK)
        def _zero(i):
            for j in range(D // 16):
                rows_a[i, pl.ds(j * 16, 16)] = zv

        for k in range(ZROWS // CHUNK):
            pltpu.sync_copy(rows_a, agg.at[pl.ds(s * ZROWS + k * CHUNK, CHUNK)])
        plsc.subcore_barrier()

    def _start_gather(j, buf, sem):
        pltpu.async_copy(h_hbm.at[src_v.at[j]], buf, sem)

    def _wait_gather(buf, sem):
        pltpu.make_async_copy(h_hbm.at[pl.ds(0, CHUNK)], buf, sem).wait()

    with jax.named_scope("agg_edges"):
        for g in range(NG):
            gbase = tile_chunk_base + g * G
            pltpu.sync_copy(src_hbm.at[pl.ds(gbase, G)], src_v)
            pltpu.sync_copy(dst_hbm.at[pl.ds(gbase, G)], dst_v)
            _start_gather(0, rows_a, sem_a)

            @pl.loop(0, G, step=2)
            def _edges(j):
                _start_gather(j + 1, rows_b, sem_b)
                _wait_gather(rows_a, sem_a)
                pltpu.sync_copy(rows_a, agg.at[dst_v.at[j]], add=True)

                @pl.when(j + 2 < G)
                def _():
                    _start_gather(j + 2, rows_a, sem_a)

                _wait_gather(rows_b, sem_b)
                pltpu.sync_copy(rows_b, agg.at[dst_v.at[j + 1]], add=True)

    with jax.named_scope("agg_wout"):
        plsc.subcore_barrier()
        pltpu.sync_copy(
            agg.at[pl.ds(s * ZROWS, ZROWS)],
            out_hbm.at[c, pl.ds(s * ZROWS, ZROWS)],
        )


BLK = 1000


def _mlp_body(h_ref, p0_ref, p1_ref, wa_ref, ba_ref, wb_ref, bb_ref, o_ref):
    t = h_ref[...] + p0_ref[...] + p1_ref[...]
    t = jnp.dot(t, wa_ref[...], preferred_element_type=jnp.float32) + ba_ref[...]
    o_ref[...] = jnp.dot(t, wb_ref[...], preferred_element_type=jnp.float32) + bb_ref[...]


_row_spec = pl.BlockSpec((BLK, D), lambda i: (i, 0))
_w_spec = pl.BlockSpec((D, D), lambda i: (0, 0))
_b_spec = pl.BlockSpec((1, D), lambda i: (0, 0))

_mlp = pl.pallas_call(
    _mlp_body,
    grid=(N // BLK,),
    in_specs=[_row_spec, _row_spec, _row_spec, _w_spec, _b_spec, _w_spec, _b_spec],
    out_specs=_row_spec,
    out_shape=jax.ShapeDtypeStruct((N, D), jnp.float32),
)


def _mlp_head_body(h_ref, p0_ref, p1_ref, wa_ref, ba_ref, wb_ref, bb_ref,
                   wl1_ref, bl1_ref, wl3_ref, bl3_ref, o_ref):
    t = h_ref[...] + p0_ref[...] + p1_ref[...]
    t = jnp.dot(t, wa_ref[...], preferred_element_type=jnp.float32) + ba_ref[...]
    t = jnp.dot(t, wb_ref[...], preferred_element_type=jnp.float32) + bb_ref[...]
    g = jnp.maximum(
        jnp.dot(t, wl1_ref[...], preferred_element_type=jnp.float32) + bl1_ref[...],
        0.0,
    )
    o_ref[...] = jnp.dot(g, wl3_ref[...], preferred_element_type=jnp.float32) + bl3_ref[...]


_mlp_head = pl.pallas_call(
    _mlp_head_body,
    grid=(N // BLK,),
    in_specs=[
        _row_spec, _row_spec, _row_spec,
        _w_spec, _b_spec, _w_spec, _b_spec,
        _w_spec, _b_spec,
        pl.BlockSpec((D, 1), lambda i: (0, 0)),
        pl.BlockSpec((1, 1), lambda i: (0, 0)),
    ],
    out_specs=pl.BlockSpec((BLK, 1), lambda i: (i, 0)),
    out_shape=jax.ShapeDtypeStruct((N, 1), jnp.float32),
)


def kernel(x, edge_index, W1a, b1a, W1b, b1b, W2a, b2a, W2b, b2b, Wl1, bl1, Wl3, bl3):
    src = edge_index[0].astype(jnp.int32)
    dst = edge_index[1].astype(jnp.int32)
    pad = PE - E
    pad_idx = jnp.arange(pad, dtype=jnp.int32)
    src_p = jnp.concatenate([src, pad_idx % N]).reshape(-1, CHUNK)
    dst_p = jnp.concatenate([dst, N + pad_idx % (AGG_ROWS - N)]).reshape(-1, CHUNK)

    b1a_r = b1a.reshape(1, D)
    b1b_r = b1b.reshape(1, D)
    b2a_r = b2a.reshape(1, D)
    b2b_r = b2b.reshape(1, D)
    bl1_r = bl1.reshape(1, D)
    bl3_r = bl3.reshape(1, 1)

    p = _sc_agg(x, src_p, dst_p)
    h1 = _mlp(x, p[0], p[1], W1a, b1a_r, W1b, b1b_r)
    p = _sc_agg(h1, src_p, dst_p)
    out = _mlp_head(h1, p[0], p[1], W2a, b2a_r, W2b, b2b_r, Wl1, bl1_r, Wl3, bl3_r)
    return out
